# baseline (speedup 1.0000x reference)
import functools

import jax
import jax.numpy as jnp
from jax import lax
from jax.experimental import pallas as pl
from jax.experimental.pallas import tpu as pltpu
from jax.experimental.pallas import tpu_sc as plsc

N = 10000
E = 160000
N_GRAPHS = 64
N_CLASSES = 16

NUM_CORES = 2
NUM_TILES = 16

FH = 128
EB = 125
EPT = E // NUM_TILES
RB = EPT // EB
NPAD = 10240
RPT = NPAD // NUM_TILES
IW = 16

DEPT = E // (NUM_CORES * NUM_TILES)
DRB = DEPT // EB
DEGW = 16

ROW_BLK = 2000
GRID = N // ROW_BLK


@functools.cache
def _sc_mesh():
  return plsc.VectorSubcoreMesh(
      core_axis_name="c", subcore_axis_name="s",
      num_cores=NUM_CORES, num_subcores=NUM_TILES)


@functools.cache
def _sc_degree_kernel():

  @functools.partial(
      pl.kernel,
      out_type=jax.ShapeDtypeStruct((NUM_CORES, NPAD, DEGW), jnp.float32),
      mesh=_sc_mesh(),
      scratch_types=[
          pltpu.VMEM((DRB, EB), jnp.int32),
          pltpu.VMEM((EB, DEGW), jnp.float32),
          pltpu.VMEM((RPT, DEGW), jnp.float32),
          pltpu.VMEM_SHARED((NPAD, DEGW), jnp.float32),
      ],
  )
  def k(dst_hbm, ones_hbm, zeros_hbm, out_hbm, dst_v, ones_v, wb_v, acc):
    c = lax.axis_index("c")
    s = lax.axis_index("s")
    w = c * NUM_TILES + s
    pltpu.sync_copy(dst_hbm.at[pl.ds(w * DRB, DRB)], dst_v)
    pltpu.sync_copy(ones_hbm, ones_v)
    pltpu.sync_copy(zeros_hbm, wb_v)
    pltpu.sync_copy(wb_v, acc.at[pl.ds(s * RPT, RPT)])
    plsc.subcore_barrier()

    def body(j, carry):
      pltpu.sync_copy(ones_v, acc.at[dst_v.at[j]], add=True)
      return carry

    lax.fori_loop(0, DRB, body, 0, unroll=False)
    plsc.subcore_barrier()
    pltpu.sync_copy(acc.at[pl.ds(s * RPT, RPT)], wb_v)
    pltpu.sync_copy(wb_v, out_hbm.at[c].at[pl.ds(s * RPT, RPT)])

  return k


def _sc_degree(dst2d, ones_hbm, zeros_hbm):
  return _sc_degree_kernel()(dst2d, ones_hbm, zeros_hbm)


@functools.cache
def _sc_scatter_kernel():

  @functools.partial(
      pl.kernel,
      out_type=jax.ShapeDtypeStruct((NUM_CORES, NPAD, FH), jnp.float32),
      mesh=_sc_mesh(),
      scratch_types=[
          pltpu.VMEM((IW, EB), jnp.int32),
          pltpu.VMEM((IW, EB), jnp.int32),
          pltpu.VMEM((EB, FH), jnp.float32),
          pltpu.VMEM((EB, FH), jnp.float32),
          pltpu.VMEM_SHARED((NPAD, FH), jnp.float32),
          pltpu.SemaphoreType.DMA,
          pltpu.SemaphoreType.DMA,
      ],
  )
  def k(hs_hbm, src_hbm, dst_hbm, zeros_hbm, out_hbm,
        src_v, dst_v, ga, gb, acc, sema, semb):
    c = lax.axis_index("c")
    s = lax.axis_index("s")
    pltpu.sync_copy(zeros_hbm, acc.at[pl.ds(s * RPT, RPT)])
    plsc.subcore_barrier()

    bufs = (ga, gb)
    sems = (sema, semb)

    def window(w, carry):
      base = s * RB + w * IW
      pltpu.sync_copy(src_hbm.at[pl.ds(base, IW)], src_v)
      pltpu.sync_copy(dst_hbm.at[pl.ds(base, IW)], dst_v)
      cps = {0: pltpu.async_copy(hs_hbm.at[c].at[src_v.at[0]], ga, sema)}
      for b in range(IW):
        if b + 1 < IW:
          cps[b + 1] = pltpu.async_copy(
              hs_hbm.at[c].at[src_v.at[b + 1]],
              bufs[(b + 1) % 2], sems[(b + 1) % 2])
        cps[b].wait()
        pltpu.sync_copy(bufs[b % 2], acc.at[dst_v.at[b]], add=True)
      return carry

    lax.fori_loop(0, RB // IW, window, 0, unroll=False)
    plsc.subcore_barrier()
    rows = pl.ds(s * RPT, RPT)
    pltpu.sync_copy(acc.at[rows], out_hbm.at[c].at[rows])

  return k


def _sc_scatter(hs_split, src2d, dst2d, zeros_hbm):
  return _sc_scatter_kernel()(hs_split, src2d, dst2d, zeros_hbm)


def _tc_pre(deg_raw, x, w0):
  f_in = x.shape[1]

  def body(deg_ref, x_ref, w_ref, dinv_ref, hs_ref):
    deg = deg_ref[0, :, 0:1] + 1.0
    dinv = lax.rsqrt(jnp.maximum(deg, 1e-12))
    dinv_ref[...] = dinv
    hw = jnp.dot(x_ref[...], w_ref[...], preferred_element_type=jnp.float32)
    hs = hw * dinv
    hs_ref[0] = hs[:, :FH]
    hs_ref[1] = hs[:, FH:]

  return pl.pallas_call(
      body,
      grid=(GRID,),
      in_specs=[
          pl.BlockSpec((1, ROW_BLK, FH), lambda i: (0, i, 0)),
          pl.BlockSpec((ROW_BLK, f_in), lambda i: (i, 0)),
          pl.BlockSpec((f_in, 2 * FH), lambda i: (0, 0)),
      ],
      out_specs=[
          pl.BlockSpec((ROW_BLK, 1), lambda i: (i, 0)),
          pl.BlockSpec((2, ROW_BLK, FH), lambda i: (0, i, 0)),
      ],
      out_shape=[
          jax.ShapeDtypeStruct((N, 1), jnp.float32),
          jax.ShapeDtypeStruct((2, N, FH), jnp.float32),
      ],
  )(deg_raw, x, w0)


def _tc_mid(s_split, hs_split, dinv, b, w_next):
  f = 2 * FH
  fn = w_next.shape[1]

  def body(s_ref, hs_ref, dinv_ref, b_ref, w_ref, h_ref, hsn_ref):
    sv = jnp.concatenate([s_ref[0], s_ref[1]], axis=1)
    hs = jnp.concatenate([hs_ref[0], hs_ref[1]], axis=1)
    dinv = dinv_ref[...]
    h = jnp.tanh(dinv * (sv + hs) + b_ref[...])
    h_ref[...] = h
    hw = jnp.dot(h, w_ref[...], preferred_element_type=jnp.float32)
    hsn = hw * dinv
    if fn == 2 * FH:
      hsn_ref[0] = hsn[:, :FH]
      hsn_ref[1] = hsn[:, FH:]
    else:
      hsn_ref[0] = hsn
      hsn_ref[1] = jnp.zeros_like(hsn)

  return pl.pallas_call(
      body,
      grid=(GRID,),
      in_specs=[
          pl.BlockSpec((2, ROW_BLK, FH), lambda i: (0, i, 0)),
          pl.BlockSpec((2, ROW_BLK, FH), lambda i: (0, i, 0)),
          pl.BlockSpec((ROW_BLK, 1), lambda i: (i, 0)),
          pl.BlockSpec((1, f), lambda i: (0, 0)),
          pl.BlockSpec((f, fn), lambda i: (0, 0)),
      ],
      out_specs=[
          pl.BlockSpec((ROW_BLK, f), lambda i: (i, 0)),
          pl.BlockSpec((2, ROW_BLK, FH), lambda i: (0, i, 0)),
      ],
      out_shape=[
          jax.ShapeDtypeStruct((N, f), jnp.float32),
          jax.ShapeDtypeStruct((2, N, FH), jnp.float32),
      ],
  )(s_split, hs_split, dinv, b, w_next)


def _tc_final(s3, hs3, dinv, b3, h0, h1, h2, batch_row,
              lin1_w, lin1_b, lin2_w, lin2_b):
  f3 = FH
  cat = 3 * h0.shape[1] + f3

  def body(s_ref, hs_ref, dinv_ref, b_ref, h0_ref, h1_ref, h2_ref, bt_ref,
           l1w_ref, l1b_ref, l2w_ref, l2b_ref, out_ref, pooled):
    i = pl.program_id(0)
    h3 = jnp.tanh(dinv_ref[...] * (s_ref[0] + hs_ref[0]) + b_ref[...])
    hcat = jnp.concatenate(
        [h0_ref[...], h1_ref[...], h2_ref[...], h3], axis=1)
    bt = bt_ref[0]
    gid = lax.broadcasted_iota(jnp.int32, (N_GRAPHS, 1), 0)
    onehot = (bt == gid).astype(jnp.float32)
    part = lax.dot_general(onehot, hcat, (((1,), (0,)), ((), ())),
                           preferred_element_type=jnp.float32)

    @pl.when(i == 0)
    def _():
      pooled[...] = part

    @pl.when(i > 0)
    def _():
      pooled[...] += part

    @pl.when(i == GRID - 1)
    def _():
      z = jnp.dot(pooled[...], l1w_ref[...],
                  preferred_element_type=jnp.float32) + l1b_ref[...]
      z = jnp.maximum(z, 0.0)
      z2 = jnp.dot(z, l2w_ref[...],
                   preferred_element_type=jnp.float32) + l2b_ref[...]
      m = jnp.max(z2, axis=-1, keepdims=True)
      lse = m + jnp.log(jnp.sum(jnp.exp(z2 - m), axis=-1, keepdims=True))
      out_ref[...] = z2 - lse

  return pl.pallas_call(
      body,
      grid=(GRID,),
      in_specs=[
          pl.BlockSpec((1, ROW_BLK, f3), lambda i: (0, i, 0)),
          pl.BlockSpec((1, ROW_BLK, f3), lambda i: (0, i, 0)),
          pl.BlockSpec((ROW_BLK, 1), lambda i: (i, 0)),
          pl.BlockSpec((1, f3), lambda i: (0, 0)),
          pl.BlockSpec((ROW_BLK, h0.shape[1]), lambda i: (i, 0)),
          pl.BlockSpec((ROW_BLK, h1.shape[1]), lambda i: (i, 0)),
          pl.BlockSpec((ROW_BLK, h2.shape[1]), lambda i: (i, 0)),
          pl.BlockSpec((1, 1, ROW_BLK), lambda i: (i, 0, 0)),
          pl.BlockSpec((cat, lin1_w.shape[1]), lambda i: (0, 0)),
          pl.BlockSpec((1, lin1_w.shape[1]), lambda i: (0, 0)),
          pl.BlockSpec((lin2_w.shape[0], N_CLASSES), lambda i: (0, 0)),
          pl.BlockSpec((1, N_CLASSES), lambda i: (0, 0)),
      ],
      out_specs=pl.BlockSpec((N_GRAPHS, N_CLASSES), lambda i: (0, 0)),
      out_shape=jax.ShapeDtypeStruct((N_GRAPHS, N_CLASSES), jnp.float32),
      scratch_shapes=[pltpu.VMEM((N_GRAPHS, cat), jnp.float32)],
  )(s3, hs3, dinv, b3, h0, h1, h2, batch_row,
    lin1_w, lin1_b, lin2_w, lin2_b)


def kernel(x, edge_index, batch, W0, b0, W1, b1, W2, b2, W3, b3,
           lin1_W, lin1_b, lin2_W, lin2_b):
  src2d = edge_index[0].reshape(E // EB, EB)
  dst2d = edge_index[1].reshape(E // EB, EB)
  batch_row = batch.reshape(GRID, 1, ROW_BLK)

  zeros128 = jnp.zeros((RPT, FH), jnp.float32)
  ones_table = jnp.ones((NUM_CORES, N, FH), jnp.float32)

  deg_raw = _sc_scatter(ones_table, src2d, dst2d, zeros128)
  dinv, hs0 = _tc_pre(deg_raw, x, W0)

  s0 = _sc_scatter(hs0, src2d, dst2d, zeros128)
  h0, hs1 = _tc_mid(s0, hs0, dinv, b0.reshape(1, -1), W1)
  s1 = _sc_scatter(hs1, src2d, dst2d, zeros128)
  h1, hs2 = _tc_mid(s1, hs1, dinv, b1.reshape(1, -1), W2)
  s2 = _sc_scatter(hs2, src2d, dst2d, zeros128)
  h2, hs3 = _tc_mid(s2, hs2, dinv, b2.reshape(1, -1), W3)
  s3 = _sc_scatter(hs3, src2d, dst2d, zeros128)

  return _tc_final(s3[0:1], hs3[0:1], dinv, b3.reshape(1, -1), h0, h1, h2,
                   batch_row, lin1_W, lin1_b.reshape(1, -1),
                   lin2_W, lin2_b.reshape(1, -1))


# speedup vs baseline: 1.1205x; 1.1205x over previous
import functools

import jax
import jax.numpy as jnp
from jax import lax
from jax.experimental import pallas as pl
from jax.experimental.pallas import tpu as pltpu
from jax.experimental.pallas import tpu_sc as plsc

N = 10000
E = 160000
N_GRAPHS = 64
N_CLASSES = 16

NUM_CORES = 2
NUM_TILES = 16

FH = 128
EB = 125
EPT = E // NUM_TILES
RB = EPT // EB
NPAD = 10240
RPT = NPAD // NUM_TILES
IW = 16

DEPT = E // (NUM_CORES * NUM_TILES)
DRB = DEPT // EB

DEPT = E // (NUM_CORES * NUM_TILES)
DRB = DEPT // EB
DEGW = 16

ROW_BLK = 2000
GRID = N // ROW_BLK


@functools.cache
def _sc_mesh():
  return plsc.VectorSubcoreMesh(
      core_axis_name="c", subcore_axis_name="s",
      num_cores=NUM_CORES, num_subcores=NUM_TILES)


@functools.cache
def _sc_degree_kernel():

  @functools.partial(
      pl.kernel,
      out_type=jax.ShapeDtypeStruct((NUM_CORES, NPAD, DEGW), jnp.float32),
      mesh=_sc_mesh(),
      scratch_types=[
          pltpu.VMEM((DRB, EB), jnp.int32),
          pltpu.VMEM((EB, DEGW), jnp.float32),
          pltpu.VMEM((RPT, DEGW), jnp.float32),
          pltpu.VMEM_SHARED((NPAD, DEGW), jnp.float32),
      ],
  )
  def k(dst_hbm, ones_hbm, zeros_hbm, out_hbm, dst_v, ones_v, wb_v, acc):
    c = lax.axis_index("c")
    s = lax.axis_index("s")
    w = c * NUM_TILES + s
    pltpu.sync_copy(dst_hbm.at[pl.ds(w * DRB, DRB)], dst_v)
    pltpu.sync_copy(ones_hbm, ones_v)
    pltpu.sync_copy(zeros_hbm, wb_v)
    pltpu.sync_copy(wb_v, acc.at[pl.ds(s * RPT, RPT)])
    plsc.subcore_barrier()

    def body(j, carry):
      pltpu.sync_copy(ones_v, acc.at[dst_v.at[j]], add=True)
      return carry

    lax.fori_loop(0, DRB, body, 0, unroll=False)
    plsc.subcore_barrier()
    pltpu.sync_copy(acc.at[pl.ds(s * RPT, RPT)], wb_v)
    pltpu.sync_copy(wb_v, out_hbm.at[c].at[pl.ds(s * RPT, RPT)])

  return k


def _sc_degree(dst2d, ones_hbm, zeros_hbm):
  return _sc_degree_kernel()(dst2d, ones_hbm, zeros_hbm)


@functools.cache
def _sc_degree_kernel():

  @functools.partial(
      pl.kernel,
      out_type=jax.ShapeDtypeStruct((NUM_CORES, NPAD, FH), jnp.float32),
      mesh=_sc_mesh(),
      scratch_types=[
          pltpu.VMEM((DRB, EB), jnp.int32),
          pltpu.VMEM((EB, FH), jnp.float32),
          pltpu.VMEM_SHARED((NPAD, FH), jnp.float32),
      ],
  )
  def k(dst_hbm, ones_hbm, zeros_hbm, out_hbm, dst_v, ones_v, acc):
    c = lax.axis_index("c")
    s = lax.axis_index("s")
    w = c * NUM_TILES + s
    pltpu.sync_copy(dst_hbm.at[pl.ds(w * DRB, DRB)], dst_v)
    pltpu.sync_copy(ones_hbm, ones_v)
    pltpu.sync_copy(zeros_hbm, acc.at[pl.ds(s * RPT, RPT)])
    plsc.subcore_barrier()

    def body(j, carry):
      pltpu.sync_copy(ones_v, acc.at[dst_v.at[j]], add=True)
      return carry

    lax.fori_loop(0, DRB, body, 0, unroll=False)
    plsc.subcore_barrier()
    rows = pl.ds(s * RPT, RPT)
    pltpu.sync_copy(acc.at[rows], out_hbm.at[c].at[rows])

  return k


def _sc_degree(dst2d, ones_hbm, zeros_hbm):
  return _sc_degree_kernel()(dst2d, ones_hbm, zeros_hbm)


@functools.cache
def _sc_scatter_kernel():

  @functools.partial(
      pl.kernel,
      out_type=jax.ShapeDtypeStruct((NUM_CORES, NPAD, FH), jnp.float32),
      mesh=_sc_mesh(),
      scratch_types=[
          pltpu.VMEM((IW, EB), jnp.int32),
          pltpu.VMEM((IW, EB), jnp.int32),
          pltpu.VMEM((EB, FH), jnp.float32),
          pltpu.VMEM((EB, FH), jnp.float32),
          pltpu.VMEM_SHARED((NPAD, FH), jnp.float32),
          pltpu.SemaphoreType.DMA,
          pltpu.SemaphoreType.DMA,
      ],
  )
  def k(hs_hbm, src_hbm, dst_hbm, zeros_hbm, out_hbm,
        src_v, dst_v, ga, gb, acc, sema, semb):
    c = lax.axis_index("c")
    s = lax.axis_index("s")
    pltpu.sync_copy(zeros_hbm, acc.at[pl.ds(s * RPT, RPT)])
    plsc.subcore_barrier()

    bufs = (ga, gb)
    sems = (sema, semb)

    def window(w, carry):
      base = s * RB + w * IW
      pltpu.sync_copy(src_hbm.at[pl.ds(base, IW)], src_v)
      pltpu.sync_copy(dst_hbm.at[pl.ds(base, IW)], dst_v)
      cps = {0: pltpu.async_copy(hs_hbm.at[c].at[src_v.at[0]], ga, sema)}
      for b in range(IW):
        if b + 1 < IW:
          cps[b + 1] = pltpu.async_copy(
              hs_hbm.at[c].at[src_v.at[b + 1]],
              bufs[(b + 1) % 2], sems[(b + 1) % 2])
        cps[b].wait()
        pltpu.sync_copy(bufs[b % 2], acc.at[dst_v.at[b]], add=True)
      return carry

    lax.fori_loop(0, RB // IW, window, 0, unroll=False)
    plsc.subcore_barrier()
    rows = pl.ds(s * RPT, RPT)
    pltpu.sync_copy(acc.at[rows], out_hbm.at[c].at[rows])

  return k


def _sc_scatter(hs_split, src2d, dst2d, zeros_hbm):
  return _sc_scatter_kernel()(hs_split, src2d, dst2d, zeros_hbm)


def _tc_pre(deg_raw, x, w0):
  f_in = x.shape[1]

  def body(deg_ref, x_ref, w_ref, dinv_ref, hs_ref):
    deg = deg_ref[0, :, 0:1] + deg_ref[1, :, 0:1] + 1.0
    dinv = lax.rsqrt(jnp.maximum(deg, 1e-12))
    dinv_ref[...] = dinv
    hw = jnp.dot(x_ref[...], w_ref[...], preferred_element_type=jnp.float32)
    hs = hw * dinv
    hs_ref[0] = hs[:, :FH]
    hs_ref[1] = hs[:, FH:]

  return pl.pallas_call(
      body,
      grid=(GRID,),
      in_specs=[
          pl.BlockSpec((2, ROW_BLK, FH), lambda i: (0, i, 0)),
          pl.BlockSpec((ROW_BLK, f_in), lambda i: (i, 0)),
          pl.BlockSpec((f_in, 2 * FH), lambda i: (0, 0)),
      ],
      out_specs=[
          pl.BlockSpec((ROW_BLK, 1), lambda i: (i, 0)),
          pl.BlockSpec((2, ROW_BLK, FH), lambda i: (0, i, 0)),
      ],
      out_shape=[
          jax.ShapeDtypeStruct((N, 1), jnp.float32),
          jax.ShapeDtypeStruct((2, N, FH), jnp.float32),
      ],
  )(deg_raw, x, w0)


def _tc_mid(s_split, hs_split, dinv, b, w_next):
  f = 2 * FH
  fn = w_next.shape[1]

  def body(s_ref, hs_ref, dinv_ref, b_ref, w_ref, h_ref, hsn_ref):
    sv = jnp.concatenate([s_ref[0], s_ref[1]], axis=1)
    hs = jnp.concatenate([hs_ref[0], hs_ref[1]], axis=1)
    dinv = dinv_ref[...]
    h = jnp.tanh(dinv * (sv + hs) + b_ref[...])
    h_ref[...] = h
    hw = jnp.dot(h, w_ref[...], preferred_element_type=jnp.float32)
    hsn = hw * dinv
    if fn == 2 * FH:
      hsn_ref[0] = hsn[:, :FH]
      hsn_ref[1] = hsn[:, FH:]
    else:
      hsn_ref[0] = hsn
      hsn_ref[1] = jnp.zeros_like(hsn)

  return pl.pallas_call(
      body,
      grid=(GRID,),
      in_specs=[
          pl.BlockSpec((2, ROW_BLK, FH), lambda i: (0, i, 0)),
          pl.BlockSpec((2, ROW_BLK, FH), lambda i: (0, i, 0)),
          pl.BlockSpec((ROW_BLK, 1), lambda i: (i, 0)),
          pl.BlockSpec((1, f), lambda i: (0, 0)),
          pl.BlockSpec((f, fn), lambda i: (0, 0)),
      ],
      out_specs=[
          pl.BlockSpec((ROW_BLK, f), lambda i: (i, 0)),
          pl.BlockSpec((2, ROW_BLK, FH), lambda i: (0, i, 0)),
      ],
      out_shape=[
          jax.ShapeDtypeStruct((N, f), jnp.float32),
          jax.ShapeDtypeStruct((2, N, FH), jnp.float32),
      ],
  )(s_split, hs_split, dinv, b, w_next)


def _tc_final(s3, hs3, dinv, b3, h0, h1, h2, batch_row,
              lin1_w, lin1_b, lin2_w, lin2_b):
  f3 = FH
  cat = 3 * h0.shape[1] + f3

  def body(s_ref, hs_ref, dinv_ref, b_ref, h0_ref, h1_ref, h2_ref, bt_ref,
           l1w_ref, l1b_ref, l2w_ref, l2b_ref, out_ref, pooled):
    i = pl.program_id(0)
    h3 = jnp.tanh(dinv_ref[...] * (s_ref[0] + hs_ref[0]) + b_ref[...])
    hcat = jnp.concatenate(
        [h0_ref[...], h1_ref[...], h2_ref[...], h3], axis=1)
    bt = bt_ref[0]
    gid = lax.broadcasted_iota(jnp.int32, (N_GRAPHS, 1), 0)
    onehot = (bt == gid).astype(jnp.float32)
    part = lax.dot_general(onehot, hcat, (((1,), (0,)), ((), ())),
                           preferred_element_type=jnp.float32)

    @pl.when(i == 0)
    def _():
      pooled[...] = part

    @pl.when(i > 0)
    def _():
      pooled[...] += part

    @pl.when(i == GRID - 1)
    def _():
      z = jnp.dot(pooled[...], l1w_ref[...],
                  preferred_element_type=jnp.float32) + l1b_ref[...]
      z = jnp.maximum(z, 0.0)
      z2 = jnp.dot(z, l2w_ref[...],
                   preferred_element_type=jnp.float32) + l2b_ref[...]
      m = jnp.max(z2, axis=-1, keepdims=True)
      lse = m + jnp.log(jnp.sum(jnp.exp(z2 - m), axis=-1, keepdims=True))
      out_ref[...] = z2 - lse

  return pl.pallas_call(
      body,
      grid=(GRID,),
      in_specs=[
          pl.BlockSpec((1, ROW_BLK, f3), lambda i: (0, i, 0)),
          pl.BlockSpec((1, ROW_BLK, f3), lambda i: (0, i, 0)),
          pl.BlockSpec((ROW_BLK, 1), lambda i: (i, 0)),
          pl.BlockSpec((1, f3), lambda i: (0, 0)),
          pl.BlockSpec((ROW_BLK, h0.shape[1]), lambda i: (i, 0)),
          pl.BlockSpec((ROW_BLK, h1.shape[1]), lambda i: (i, 0)),
          pl.BlockSpec((ROW_BLK, h2.shape[1]), lambda i: (i, 0)),
          pl.BlockSpec((1, 1, ROW_BLK), lambda i: (i, 0, 0)),
          pl.BlockSpec((cat, lin1_w.shape[1]), lambda i: (0, 0)),
          pl.BlockSpec((1, lin1_w.shape[1]), lambda i: (0, 0)),
          pl.BlockSpec((lin2_w.shape[0], N_CLASSES), lambda i: (0, 0)),
          pl.BlockSpec((1, N_CLASSES), lambda i: (0, 0)),
      ],
      out_specs=pl.BlockSpec((N_GRAPHS, N_CLASSES), lambda i: (0, 0)),
      out_shape=jax.ShapeDtypeStruct((N_GRAPHS, N_CLASSES), jnp.float32),
      scratch_shapes=[pltpu.VMEM((N_GRAPHS, cat), jnp.float32)],
  )(s3, hs3, dinv, b3, h0, h1, h2, batch_row,
    lin1_w, lin1_b, lin2_w, lin2_b)


def kernel(x, edge_index, batch, W0, b0, W1, b1, W2, b2, W3, b3,
           lin1_W, lin1_b, lin2_W, lin2_b):
  src2d = edge_index[0].reshape(E // EB, EB)
  dst2d = edge_index[1].reshape(E // EB, EB)
  batch_row = batch.reshape(GRID, 1, ROW_BLK)

  zeros128 = jnp.zeros((RPT, FH), jnp.float32)
  ones_deg = jnp.ones((EB, FH), jnp.float32)

  deg_raw = _sc_degree(dst2d, ones_deg, zeros128)
  dinv, hs0 = _tc_pre(deg_raw, x, W0)

  s0 = _sc_scatter(hs0, src2d, dst2d, zeros128)
  h0, hs1 = _tc_mid(s0, hs0, dinv, b0.reshape(1, -1), W1)
  s1 = _sc_scatter(hs1, src2d, dst2d, zeros128)
  h1, hs2 = _tc_mid(s1, hs1, dinv, b1.reshape(1, -1), W2)
  s2 = _sc_scatter(hs2, src2d, dst2d, zeros128)
  h2, hs3 = _tc_mid(s2, hs2, dinv, b2.reshape(1, -1), W3)
  s3 = _sc_scatter(hs3, src2d, dst2d, zeros128)

  return _tc_final(s3[0:1], hs3[0:1], dinv, b3.reshape(1, -1), h0, h1, h2,
                   batch_row, lin1_W, lin1_b.reshape(1, -1),
                   lin2_W, lin2_b.reshape(1, -1))


# speedup vs baseline: 1.1972x; 1.0685x over previous
import functools

import jax
import jax.numpy as jnp
from jax import lax
from jax.experimental import pallas as pl
from jax.experimental.pallas import tpu as pltpu
from jax.experimental.pallas import tpu_sc as plsc

N = 10000
E = 160000
N_GRAPHS = 64
N_CLASSES = 16

NUM_CORES = 2
NUM_TILES = 16

FH = 128
EB = 125
EPT = E // NUM_TILES
RB = EPT // EB
NPAD = 10240
RPT = NPAD // NUM_TILES
IW = 8

DEPT = E // (NUM_CORES * NUM_TILES)
DRB = DEPT // EB

DEPT = E // (NUM_CORES * NUM_TILES)
DRB = DEPT // EB
DEGW = 16

ROW_BLK = 2000
GRID = N // ROW_BLK


@functools.cache
def _sc_mesh():
  return plsc.VectorSubcoreMesh(
      core_axis_name="c", subcore_axis_name="s",
      num_cores=NUM_CORES, num_subcores=NUM_TILES)


@functools.cache
def _sc_degree_kernel():

  @functools.partial(
      pl.kernel,
      out_type=jax.ShapeDtypeStruct((NUM_CORES, NPAD, DEGW), jnp.float32),
      mesh=_sc_mesh(),
      scratch_types=[
          pltpu.VMEM((DRB, EB), jnp.int32),
          pltpu.VMEM((EB, DEGW), jnp.float32),
          pltpu.VMEM((RPT, DEGW), jnp.float32),
          pltpu.VMEM_SHARED((NPAD, DEGW), jnp.float32),
      ],
  )
  def k(dst_hbm, ones_hbm, zeros_hbm, out_hbm, dst_v, ones_v, wb_v, acc):
    c = lax.axis_index("c")
    s = lax.axis_index("s")
    w = c * NUM_TILES + s
    pltpu.sync_copy(dst_hbm.at[pl.ds(w * DRB, DRB)], dst_v)
    pltpu.sync_copy(ones_hbm, ones_v)
    pltpu.sync_copy(zeros_hbm, wb_v)
    pltpu.sync_copy(wb_v, acc.at[pl.ds(s * RPT, RPT)])
    plsc.subcore_barrier()

    def body(j, carry):
      pltpu.sync_copy(ones_v, acc.at[dst_v.at[j]], add=True)
      return carry

    lax.fori_loop(0, DRB, body, 0, unroll=False)
    plsc.subcore_barrier()
    pltpu.sync_copy(acc.at[pl.ds(s * RPT, RPT)], wb_v)
    pltpu.sync_copy(wb_v, out_hbm.at[c].at[pl.ds(s * RPT, RPT)])

  return k


def _sc_degree(dst2d, ones_hbm, zeros_hbm):
  return _sc_degree_kernel()(dst2d, ones_hbm, zeros_hbm)


@functools.cache
def _sc_degree_kernel():

  @functools.partial(
      pl.kernel,
      out_type=jax.ShapeDtypeStruct((NUM_CORES, NPAD, FH), jnp.float32),
      mesh=_sc_mesh(),
      scratch_types=[
          pltpu.VMEM((DRB, EB), jnp.int32),
          pltpu.VMEM((EB, FH), jnp.float32),
          pltpu.VMEM_SHARED((NPAD, FH), jnp.float32),
      ],
  )
  def k(dst_hbm, ones_hbm, zeros_hbm, out_hbm, dst_v, ones_v, acc):
    c = lax.axis_index("c")
    s = lax.axis_index("s")
    w = c * NUM_TILES + s
    pltpu.sync_copy(dst_hbm.at[pl.ds(w * DRB, DRB)], dst_v)
    pltpu.sync_copy(ones_hbm, ones_v)
    pltpu.sync_copy(zeros_hbm, acc.at[pl.ds(s * RPT, RPT)])
    plsc.subcore_barrier()

    def body(j, carry):
      pltpu.sync_copy(ones_v, acc.at[dst_v.at[j]], add=True)
      return carry

    lax.fori_loop(0, DRB, body, 0, unroll=False)
    plsc.subcore_barrier()
    rows = pl.ds(s * RPT, RPT)
    pltpu.sync_copy(acc.at[rows], out_hbm.at[c].at[rows])

  return k


def _sc_degree(dst2d, ones_hbm, zeros_hbm):
  return _sc_degree_kernel()(dst2d, ones_hbm, zeros_hbm)


@functools.cache
def _sc_scatter_kernel():

  @functools.partial(
      pl.kernel,
      out_type=jax.ShapeDtypeStruct((NUM_CORES, NPAD, FH), jnp.float32),
      mesh=_sc_mesh(),
      scratch_types=[
          pltpu.VMEM((IW, EB), jnp.int32),
          pltpu.VMEM((IW, EB), jnp.int32),
          pltpu.VMEM((IW, EB), jnp.int32),
          pltpu.VMEM((IW, EB), jnp.int32),
          pltpu.VMEM((EB, FH), jnp.float32),
          pltpu.VMEM((EB, FH), jnp.float32),
          pltpu.VMEM_SHARED((NPAD, FH), jnp.float32),
          pltpu.SemaphoreType.DMA,
          pltpu.SemaphoreType.DMA,
          pltpu.SemaphoreType.DMA,
      ],
  )
  def k(hs_hbm, src_hbm, dst_hbm, zeros_hbm, out_hbm,
        src_a, dst_a, src_b, dst_b, ga, gb, acc, sema, semb, semi):
    c = lax.axis_index("c")
    s = lax.axis_index("s")
    pltpu.sync_copy(zeros_hbm, acc.at[pl.ds(s * RPT, RPT)])
    plsc.subcore_barrier()

    bufs = (ga, gb)
    sems = (sema, semb)
    base = s * RB
    nw = RB // IW

    def iwin(arr, w):
      return arr.at[pl.ds(base + w * IW, IW)]

    def fire(sv, b):
      return pltpu.async_copy(hs_hbm.at[c].at[sv.at[b]], bufs[b % 2],
                              sems[b % 2])

    def wait0(sv):
      pltpu.make_async_copy(hs_hbm.at[c].at[sv.at[0]], ga, sema).wait()

    def window(sv, dv, sv_next, cp_first, idx_waits):
      cps = {0: cp_first} if cp_first is not None else {}
      nxt = None
      for b in range(IW):
        if b + 1 < IW:
          cps[b + 1] = fire(sv, b + 1)
        else:
          for iw_ in idx_waits:
            iw_.wait()
          nxt = fire(sv_next, 0)
        if b in cps:
          cps[b].wait()
        else:
          wait0(sv)
        pltpu.sync_copy(bufs[b % 2], acc.at[dv.at[b]], add=True)
      return nxt

    pltpu.sync_copy(iwin(src_hbm, 0), src_a)
    pltpu.sync_copy(iwin(dst_hbm, 0), dst_a)
    fire(src_a, 0)

    def dbl(t, carry):
      w1 = 2 * t + 1
      w2 = jnp.minimum(2 * t + 2, nw - 1)
      i1 = (pltpu.async_copy(iwin(src_hbm, w1), src_b, semi),
            pltpu.async_copy(iwin(dst_hbm, w1), dst_b, semi))
      cp = window(src_a, dst_a, src_b, None, i1)
      i2 = (pltpu.async_copy(iwin(src_hbm, w2), src_a, semi),
            pltpu.async_copy(iwin(dst_hbm, w2), dst_a, semi))
      window(src_b, dst_b, src_a, cp, i2)
      return carry

    lax.fori_loop(0, nw // 2, dbl, 0, unroll=False)
    wait0(src_a)
    plsc.subcore_barrier()
    rows = pl.ds(s * RPT, RPT)
    pltpu.sync_copy(acc.at[rows], out_hbm.at[c].at[rows])

  return k


def _sc_scatter(hs_split, src2d, dst2d, zeros_hbm):
  return _sc_scatter_kernel()(hs_split, src2d, dst2d, zeros_hbm)


def _tc_pre(deg_raw, x, w0):
  f_in = x.shape[1]

  def body(deg_ref, x_ref, w_ref, dinv_ref, hs_ref):
    deg = deg_ref[0, :, 0:1] + deg_ref[1, :, 0:1] + 1.0
    dinv = lax.rsqrt(jnp.maximum(deg, 1e-12))
    dinv_ref[...] = dinv
    hw = jnp.dot(x_ref[...], w_ref[...], preferred_element_type=jnp.float32)
    hs = hw * dinv
    hs_ref[0] = hs[:, :FH]
    hs_ref[1] = hs[:, FH:]

  return pl.pallas_call(
      body,
      grid=(GRID,),
      in_specs=[
          pl.BlockSpec((2, ROW_BLK, FH), lambda i: (0, i, 0)),
          pl.BlockSpec((ROW_BLK, f_in), lambda i: (i, 0)),
          pl.BlockSpec((f_in, 2 * FH), lambda i: (0, 0)),
      ],
      out_specs=[
          pl.BlockSpec((ROW_BLK, 1), lambda i: (i, 0)),
          pl.BlockSpec((2, ROW_BLK, FH), lambda i: (0, i, 0)),
      ],
      out_shape=[
          jax.ShapeDtypeStruct((N, 1), jnp.float32),
          jax.ShapeDtypeStruct((2, N, FH), jnp.float32),
      ],
  )(deg_raw, x, w0)


def _tc_mid(s_split, hs_split, dinv, b, w_next):
  f = 2 * FH
  fn = w_next.shape[1]

  def body(s_ref, hs_ref, dinv_ref, b_ref, w_ref, h_ref, hsn_ref):
    sv = jnp.concatenate([s_ref[0], s_ref[1]], axis=1)
    hs = jnp.concatenate([hs_ref[0], hs_ref[1]], axis=1)
    dinv = dinv_ref[...]
    h = jnp.tanh(dinv * (sv + hs) + b_ref[...])
    h_ref[...] = h
    hw = jnp.dot(h, w_ref[...], preferred_element_type=jnp.float32)
    hsn = hw * dinv
    if fn == 2 * FH:
      hsn_ref[0] = hsn[:, :FH]
      hsn_ref[1] = hsn[:, FH:]
    else:
      hsn_ref[0] = hsn
      hsn_ref[1] = jnp.zeros_like(hsn)

  return pl.pallas_call(
      body,
      grid=(GRID,),
      in_specs=[
          pl.BlockSpec((2, ROW_BLK, FH), lambda i: (0, i, 0)),
          pl.BlockSpec((2, ROW_BLK, FH), lambda i: (0, i, 0)),
          pl.BlockSpec((ROW_BLK, 1), lambda i: (i, 0)),
          pl.BlockSpec((1, f), lambda i: (0, 0)),
          pl.BlockSpec((f, fn), lambda i: (0, 0)),
      ],
      out_specs=[
          pl.BlockSpec((ROW_BLK, f), lambda i: (i, 0)),
          pl.BlockSpec((2, ROW_BLK, FH), lambda i: (0, i, 0)),
      ],
      out_shape=[
          jax.ShapeDtypeStruct((N, f), jnp.float32),
          jax.ShapeDtypeStruct((2, N, FH), jnp.float32),
      ],
  )(s_split, hs_split, dinv, b, w_next)


def _tc_final(s3, hs3, dinv, b3, h0, h1, h2, batch_row,
              lin1_w, lin1_b, lin2_w, lin2_b):
  f3 = FH
  cat = 3 * h0.shape[1] + f3

  def body(s_ref, hs_ref, dinv_ref, b_ref, h0_ref, h1_ref, h2_ref, bt_ref,
           l1w_ref, l1b_ref, l2w_ref, l2b_ref, out_ref, pooled):
    i = pl.program_id(0)
    h3 = jnp.tanh(dinv_ref[...] * (s_ref[0] + hs_ref[0]) + b_ref[...])
    hcat = jnp.concatenate(
        [h0_ref[...], h1_ref[...], h2_ref[...], h3], axis=1)
    bt = bt_ref[0]
    gid = lax.broadcasted_iota(jnp.int32, (N_GRAPHS, 1), 0)
    onehot = (bt == gid).astype(jnp.float32)
    part = lax.dot_general(onehot, hcat, (((1,), (0,)), ((), ())),
                           preferred_element_type=jnp.float32)

    @pl.when(i == 0)
    def _():
      pooled[...] = part

    @pl.when(i > 0)
    def _():
      pooled[...] += part

    @pl.when(i == GRID - 1)
    def _():
      z = jnp.dot(pooled[...], l1w_ref[...],
                  preferred_element_type=jnp.float32) + l1b_ref[...]
      z = jnp.maximum(z, 0.0)
      z2 = jnp.dot(z, l2w_ref[...],
                   preferred_element_type=jnp.float32) + l2b_ref[...]
      m = jnp.max(z2, axis=-1, keepdims=True)
      lse = m + jnp.log(jnp.sum(jnp.exp(z2 - m), axis=-1, keepdims=True))
      out_ref[...] = z2 - lse

  return pl.pallas_call(
      body,
      grid=(GRID,),
      in_specs=[
          pl.BlockSpec((1, ROW_BLK, f3), lambda i: (0, i, 0)),
          pl.BlockSpec((1, ROW_BLK, f3), lambda i: (0, i, 0)),
          pl.BlockSpec((ROW_BLK, 1), lambda i: (i, 0)),
          pl.BlockSpec((1, f3), lambda i: (0, 0)),
          pl.BlockSpec((ROW_BLK, h0.shape[1]), lambda i: (i, 0)),
          pl.BlockSpec((ROW_BLK, h1.shape[1]), lambda i: (i, 0)),
          pl.BlockSpec((ROW_BLK, h2.shape[1]), lambda i: (i, 0)),
          pl.BlockSpec((1, 1, ROW_BLK), lambda i: (i, 0, 0)),
          pl.BlockSpec((cat, lin1_w.shape[1]), lambda i: (0, 0)),
          pl.BlockSpec((1, lin1_w.shape[1]), lambda i: (0, 0)),
          pl.BlockSpec((lin2_w.shape[0], N_CLASSES), lambda i: (0, 0)),
          pl.BlockSpec((1, N_CLASSES), lambda i: (0, 0)),
      ],
      out_specs=pl.BlockSpec((N_GRAPHS, N_CLASSES), lambda i: (0, 0)),
      out_shape=jax.ShapeDtypeStruct((N_GRAPHS, N_CLASSES), jnp.float32),
      scratch_shapes=[pltpu.VMEM((N_GRAPHS, cat), jnp.float32)],
  )(s3, hs3, dinv, b3, h0, h1, h2, batch_row,
    lin1_w, lin1_b, lin2_w, lin2_b)


def kernel(x, edge_index, batch, W0, b0, W1, b1, W2, b2, W3, b3,
           lin1_W, lin1_b, lin2_W, lin2_b):
  src2d = edge_index[0].reshape(E // EB, EB)
  dst2d = edge_index[1].reshape(E // EB, EB)
  batch_row = batch.reshape(GRID, 1, ROW_BLK)

  zeros128 = jnp.zeros((RPT, FH), jnp.float32)
  ones_deg = jnp.ones((EB, FH), jnp.float32)

  deg_raw = _sc_degree(dst2d, ones_deg, zeros128)
  dinv, hs0 = _tc_pre(deg_raw, x, W0)

  s0 = _sc_scatter(hs0, src2d, dst2d, zeros128)
  h0, hs1 = _tc_mid(s0, hs0, dinv, b0.reshape(1, -1), W1)
  s1 = _sc_scatter(hs1, src2d, dst2d, zeros128)
  h1, hs2 = _tc_mid(s1, hs1, dinv, b1.reshape(1, -1), W2)
  s2 = _sc_scatter(hs2, src2d, dst2d, zeros128)
  h2, hs3 = _tc_mid(s2, hs2, dinv, b2.reshape(1, -1), W3)
  s3 = _sc_scatter(hs3, src2d, dst2d, zeros128)

  return _tc_final(s3[0:1], hs3[0:1], dinv, b3.reshape(1, -1), h0, h1, h2,
                   batch_row, lin1_W, lin1_b.reshape(1, -1),
                   lin2_W, lin2_b.reshape(1, -1))
